# baseline (speedup 1.0000x reference)
import functools

import jax
import jax.numpy as jnp
from jax import lax
from jax.experimental import pallas as pl
from jax.experimental.pallas import tpu as pltpu
from jax.experimental.pallas import tpu_sc as plsc

BATCH = 4096
SEQ = 200
EMBED = 32
NUM_INDICES = BATCH * SEQ

NC, NS = 2, 16
NW = NC * NS

BCHUNKS = BATCH // 128
UNITS = SEQ * BCHUNKS
UNITS_PER_W = UNITS // NW
ET = EMBED // 8


def _table_add_body(t_ref, p_ref, o_ref):
    o_ref[...] = t_ref[...] + p_ref[...]


def _combined_table(token_table, pos_table):
    v, d = token_table.shape
    blk = 5000
    return pl.pallas_call(
        _table_add_body,
        grid=(v // blk,),
        in_specs=[pl.BlockSpec((blk, d), lambda i: (i, 0))] * 2,
        out_specs=pl.BlockSpec((blk, d), lambda i: (i, 0)),
        out_shape=jax.ShapeDtypeStruct((v, d), jnp.float32),
    )(token_table, pos_table)


_MESH = plsc.VectorSubcoreMesh(core_axis_name="c", subcore_axis_name="s")


@functools.partial(
    pl.kernel,
    out_type=jax.ShapeDtypeStruct((SEQ, ET, BCHUNKS, 8, 128), jnp.float32),
    mesh=_MESH,
    scratch_types=[
        pltpu.VMEM((UNITS_PER_W, 128), jnp.int32),
        pltpu.VMEM((128, EMBED), jnp.float32),
        pltpu.VMEM((128, EMBED), jnp.float32),
        pltpu.VMEM((ET, 8, 128), jnp.float32),
        pltpu.VMEM((ET, 8, 128), jnp.float32),
        pltpu.SemaphoreType.DMA,
        pltpu.SemaphoreType.DMA,
        pltpu.SemaphoreType.DMA,
        pltpu.SemaphoreType.DMA,
    ],
    compiler_params=pltpu.CompilerParams(
        use_tc_tiling_on_sc=False, needs_layout_passes=False
    ),
)
def _sc_gather_t(table_hbm, idx_hbm, out_hbm, idx_all, rows_a, rows_b, t_a,
                 t_b, sem_ga, sem_gb, sem_oa, sem_ob):
    wid = lax.axis_index("s") * NC + lax.axis_index("c")
    u0 = wid * UNITS_PER_W
    iota = lax.iota(jnp.int32, 16)

    pltpu.sync_copy(idx_hbm.at[pl.ds(u0, UNITS_PER_W)], idx_all)
    pltpu.async_copy(table_hbm.at[idx_all.at[0]], rows_a, sem_ga)

    def transpose_unit(rows_v, t_v):
        for e in range(EMBED):
            et, e8 = divmod(e, 8)
            col = jnp.full((16,), e, jnp.int32)
            for k in range(8):
                vals = plsc.load_gather(rows_v, [k * 16 + iota, col])
                t_v[et, e8, pl.ds(k * 16, 16)] = vals

    def substep(p, ul, rows_v, t_v, sem_g, sem_o, rows_nxt, sem_gnxt):
        @pl.when(ul + 1 < UNITS_PER_W)
        def _():
            pltpu.async_copy(table_hbm.at[idx_all.at[ul + 1]], rows_nxt,
                             sem_gnxt)

        pltpu.make_async_copy(table_hbm.at[pl.ds(0, 128)], rows_v,
                              sem_g).wait()
        @pl.when(p >= 1)
        def _():
            for et in range(ET):
                pltpu.make_async_copy(t_v.at[et], out_hbm.at[0, 0, 0],
                                      sem_o).wait()

        transpose_unit(rows_v, t_v)

        ug = u0 + ul
        s = ug // BCHUNKS
        bc = lax.rem(ug, BCHUNKS)
        for et in range(ET):
            pltpu.async_copy(t_v.at[et], out_hbm.at[s, et, bc], sem_o)

    def pair(p, carry):
        substep(p, 2 * p, rows_a, t_a, sem_ga, sem_oa, rows_b, sem_gb)
        substep(p, 2 * p + 1, rows_b, t_b, sem_gb, sem_ob, rows_a, sem_ga)
        return carry

    lax.fori_loop(0, UNITS_PER_W // 2, pair, 0)

    for t_v, sem_o in ((t_a, sem_oa), (t_b, sem_ob)):
        for et in range(ET):
            pltpu.make_async_copy(t_v.at[et], out_hbm.at[0, 0, 0],
                                  sem_o).wait()


def kernel(raw_input, token_table, pos_table):
    tok = token_table.reshape(-1, 128)
    pos = pos_table.reshape(-1, 128)
    combined = _combined_table(tok, pos).reshape(-1, EMBED)
    idx2 = raw_input.astype(jnp.int32).T.reshape(UNITS, 128)
    out5 = _sc_gather_t(combined, idx2)
    return out5.transpose(2, 4, 0, 1, 3).reshape(BATCH, SEQ, EMBED)


# speedup vs baseline: 2.3536x; 2.3536x over previous
import functools

import jax
import jax.numpy as jnp
from jax import lax
from jax.experimental import pallas as pl
from jax.experimental.pallas import tpu as pltpu
from jax.experimental.pallas import tpu_sc as plsc

BATCH = 4096
SEQ = 200
EMBED = 32
NUM_INDICES = BATCH * SEQ

NC, NS = 2, 16
NW = NC * NS
PER_WORKER = NUM_INDICES // NW

GATHER = 128
GROUP = 10
ROWS_PER_STEP = GATHER * GROUP
STEPS = PER_WORKER // ROWS_PER_STEP
IDX_ROWS_PER_W = PER_WORKER // GATHER

PACK = 128 // EMBED
OUT_ROWS = NUM_INDICES // PACK
OUT_ROWS_PER_STEP = ROWS_PER_STEP // PACK


def _table_add_body(t_ref, p_ref, o_ref):
    o_ref[...] = t_ref[...] + p_ref[...]


def _combined_table(token_table, pos_table):
    v, d = token_table.shape
    blk = 5000
    return pl.pallas_call(
        _table_add_body,
        grid=(v // blk,),
        in_specs=[pl.BlockSpec((blk, d), lambda i: (i, 0))] * 2,
        out_specs=pl.BlockSpec((blk, d), lambda i: (i, 0)),
        out_shape=jax.ShapeDtypeStruct((v, d), jnp.float32),
    )(token_table, pos_table)


_MESH = plsc.VectorSubcoreMesh(core_axis_name="c", subcore_axis_name="s")


@functools.partial(
    pl.kernel,
    out_type=jax.ShapeDtypeStruct((NUM_INDICES, 128), jnp.float32),
    mesh=_MESH,
    scratch_types=[
        pltpu.VMEM((GROUP, GATHER), jnp.int32),
        pltpu.VMEM((ROWS_PER_STEP, EMBED), jnp.float32),
        pltpu.VMEM((ROWS_PER_STEP, EMBED), jnp.float32),
        pltpu.SemaphoreType.DMA,
        pltpu.SemaphoreType.DMA,
        pltpu.SemaphoreType.DMA,
    ],
    compiler_params=pltpu.CompilerParams(use_tc_tiling_on_sc=False),
)
def _sc_gather(table_hbm, idx_hbm, out_hbm, idx_v, rows_a, rows_b, sem_g,
               sem_oa, sem_ob):
    wid = lax.axis_index("s") * NC + lax.axis_index("c")
    row0 = wid * IDX_ROWS_PER_W

    def substep(i, rows_v, sem_o):
        r = row0 + i * GROUP
        @pl.when(i >= 2)
        def _():
            pltpu.make_async_copy(
                rows_v,
                out_hbm.at[pl.ds(0, ROWS_PER_STEP), pl.ds(0, EMBED)],
                sem_o,
            ).wait()

        pltpu.sync_copy(idx_hbm.at[pl.ds(r, GROUP)], idx_v)
        copies = [
            pltpu.async_copy(
                table_hbm.at[idx_v.at[j]],
                rows_v.at[pl.ds(j * GATHER, GATHER)],
                sem_g,
            )
            for j in range(GROUP)
        ]
        for c in copies:
            c.wait()
        pltpu.async_copy(
            rows_v,
            out_hbm.at[pl.ds(r * GATHER, ROWS_PER_STEP), pl.ds(0, EMBED)],
            sem_o,
        )

    def pair(p, carry):
        substep(2 * p, rows_a, sem_oa)
        substep(2 * p + 1, rows_b, sem_ob)
        return carry

    lax.fori_loop(0, STEPS // 2, pair, 0)

    for rows_v, sem_o in ((rows_a, sem_oa), (rows_b, sem_ob)):
        pltpu.make_async_copy(
            rows_v,
            out_hbm.at[pl.ds(0, ROWS_PER_STEP), pl.ds(0, EMBED)],
            sem_o,
        ).wait()


def kernel(raw_input, token_table, pos_table):
    tok = token_table.reshape(-1, 128)
    pos = pos_table.reshape(-1, 128)
    combined = _combined_table(tok, pos).reshape(-1, EMBED)
    idx = raw_input.astype(jnp.int32).reshape(NUM_INDICES // GATHER, GATHER)
    out = _sc_gather(combined, idx)
    return out[:, :EMBED].reshape(BATCH, SEQ, EMBED)
